# baseline (speedup 1.0000x reference)
import functools
import jax
import jax.numpy as jnp
from jax import lax
from jax.experimental import pallas as pl
from jax.experimental.pallas import tpu as pltpu
from jax.experimental.pallas import tpu_sc as plsc

B = 4096
NNEG = 50
D = 32
BATA = 0.01
LAMDA = 0.01
T = float(B - 1) ** -0.5

NC = 2
NS = 16
NW = NC * NS
UPW = B // NW
NPW = UPW * NNEG
GPW = NPW // 128
NCHUNK = 10
GPC = GPW // NCHUNK
ROWS_PC = GPC * 128
PG_PC = ROWS_PC // 16


QTC = 7812
QTAIL = 64
CPC = 8


@functools.cache
def _build_sc_relayout():
  mesh = plsc.VectorSubcoreMesh(core_axis_name="c", subcore_axis_name="s")

  @functools.partial(
      pl.kernel,
      out_type=jax.ShapeDtypeStruct((250000, 128), jnp.float32),
      mesh=mesh,
      scratch_types=(
          pltpu.VMEM((32, CPC * 128), jnp.float32),
          pltpu.VMEM((CPC * 32, 128), jnp.float32),
          pltpu.VMEM((32, QTAIL), jnp.float32),
          pltpu.SemaphoreType.DMA,
      ),
      compiler_params=pltpu.CompilerParams(needs_layout_passes=False),
  )
  def _sc_relayout(qiT_hbm, out_hbm, in_v, out_v, tin_v, sem):
    w = lax.axis_index("s") * NC + lax.axis_index("c")
    lanes = jnp.arange(16, dtype=jnp.int32)
    dvec0 = lanes
    dvec1 = lanes + 16
    start = 244 * w + jnp.minimum(w, 4)
    ncols = 244 + jnp.where(w < 4, 1, 0)

    zero16 = jnp.zeros((16,), jnp.int32)

    def transpose_items(nitems, src=None):
      src_v = in_v if src is None else src

      def grp_body(jg, carry):
        j0 = jg * 8
        iv = zero16 + j0
        rb = zero16 + lax.shift_right_logical(j0, 2)
        for s in range(8):
          lo = plsc.load_gather(src_v, [dvec0, iv + s])
          hi = plsc.load_gather(src_v, [dvec1, iv + s])
          ro = rb if s < 4 else rb + 1
          cb = (s % 4) * 32
          plsc.store_scatter(out_v, [ro, cb + dvec0], lo)
          plsc.store_scatter(out_v, [ro, cb + 16 + dvec0], hi)
        return carry
      lax.fori_loop(0, nitems // 8, grp_body, 0)

    def chunk_body(k, carry):
      c0 = start + k * CPC
      cps = [pltpu.async_copy(
          qiT_hbm.at[pl.ds(kk * 8, 8), pl.ds(c0 * 128, CPC * 128)],
          in_v.at[pl.ds(kk * 8, 8), :], sem) for kk in range(4)]
      for cp in cps:
        cp.wait()
      transpose_items(CPC * 128)
      pltpu.sync_copy(out_v, out_hbm.at[pl.ds(c0 * 32, CPC * 32), :])
      return carry

    lax.fori_loop(0, ncols // CPC, chunk_body, 0)

    def rem_body(k, carry):
      c = start + (ncols // CPC) * CPC + k
      pltpu.sync_copy(qiT_hbm.at[:, pl.ds(c * 128, 128)],
                      in_v.at[:, pl.ds(0, 128)])
      transpose_items(128)
      pltpu.sync_copy(out_v.at[pl.ds(0, 32), :],
                      out_hbm.at[pl.ds(c * 32, 32), :])
      return carry

    lax.fori_loop(0, ncols % CPC, rem_body, 0)

    @pl.when(w == 4)
    def _():
      pltpu.sync_copy(qiT_hbm.at[:, pl.ds(QTC * 128, QTAIL)], tin_v)
      transpose_items(QTAIL, src=tin_v)
      pltpu.sync_copy(out_v.at[pl.ds(0, QTAIL // 4), :],
                      out_hbm.at[pl.ds(QTC * 32, QTAIL // 4), :])

  return _sc_relayout


@functools.cache
def _build_sc_fused():
  mesh = plsc.VectorSubcoreMesh(core_axis_name="c", subcore_axis_name="s")

  @functools.partial(
      pl.kernel,
      out_type=jax.ShapeDtypeStruct((NW, 1, 16), jnp.float32),
      mesh=mesh,
      scratch_types=(
          pltpu.VMEM((UPW,), jnp.int32),
          pltpu.VMEM((UPW,), jnp.int32),
          pltpu.VMEM((NPW,), jnp.int32),
          pltpu.VMEM((UPW,), jnp.int32),
          pltpu.VMEM((UPW,), jnp.int32),
          pltpu.VMEM((GPW, 128), jnp.int32),
          pltpu.VMEM((UPW, 128), jnp.float32),
          pltpu.VMEM((ROWS_PC, 128), jnp.float32),
          pltpu.VMEM((UPW,), jnp.float32),
          pltpu.VMEM((ROWS_PC,), jnp.float32),
          pltpu.VMEM((UPW,), jnp.float32),
          pltpu.VMEM((UPW,), jnp.int32),
          pltpu.VMEM((1, 16), jnp.float32),
          pltpu.SemaphoreType.DMA,
      ),
      compiler_params=pltpu.CompilerParams(needs_layout_passes=False),
  )
  def _sc_fused(users_hbm, pos_hbm, neg_hbm, pu_hbm, qi_hbm, bi_hbm,
                out_hbm,
                uidx_v, pidx_v, nidx_v, umr_v, pmr_v, nmr_v,
                urows_v, nrows_v, bpos_v, bneg_v, pscore_v, ucb_v,
                out_v, sem):
    w = lax.axis_index("s") * NC + lax.axis_index("c")
    pltpu.sync_copy(users_hbm.at[pl.ds(w * UPW, UPW)], uidx_v)
    pltpu.sync_copy(pos_hbm.at[pl.ds(w * UPW, UPW)], pidx_v)
    pltpu.sync_copy(neg_hbm.at[pl.ds(w * NPW, NPW)], nidx_v)

    lanes = jnp.arange(16, dtype=jnp.int32)
    f32z = jnp.zeros((16,), jnp.float32)

    for g in range(UPW // 16):
      cols = lanes + (g * 16)
      uit = plsc.load_gather(uidx_v, [cols])
      plsc.store_scatter(umr_v, [cols], lax.shift_right_logical(uit, 2))
      pit = plsc.load_gather(pidx_v, [cols])
      plsc.store_scatter(pmr_v, [cols], lax.shift_right_logical(pit, 2))

    def mrow_body(r, carry):
      rfull = jnp.full((16,), r, jnp.int32)
      for l in range(128 // 16):
        cols = lanes + (l * 16)
        it = plsc.load_gather(nidx_v, [r * 128 + cols])
        plsc.store_scatter(nmr_v, [rfull, cols],
                           lax.shift_right_logical(it, 2))
      return carry

    lax.fori_loop(0, GPW, mrow_body, 0)

    cp_u = pltpu.async_copy(pu_hbm.at[umr_v], urows_v, sem)
    cp_p = pltpu.async_copy(qi_hbm.at[pmr_v],
                            nrows_v.at[pl.ds(0, UPW)], sem)
    cp_b = pltpu.async_copy(bi_hbm.at[pidx_v], bpos_v, sem)
    cp_u.wait()
    cp_p.wait()
    cp_b.wait()

    acc_u2 = f32z
    acc_p2 = f32z
    acc_bi2 = f32z
    for g in range(UPW // 16):
      uids = lanes + (g * 16)
      uit = plsc.load_gather(uidx_v, [uids])
      ubase = lax.mul(lax.bitwise_and(uit, 3), 32)
      plsc.store_scatter(ucb_v, [uids], ubase)
      pit = plsc.load_gather(pidx_v, [uids])
      pbase = lax.mul(lax.bitwise_and(pit, 3), 32)
      pd = f32z
      for d in range(D):
        uv = plsc.load_gather(urows_v, [uids, ubase + d])
        pv = plsc.load_gather(nrows_v, [uids, pbase + d])
        pd = pd + uv * pv
        acc_u2 = acc_u2 + uv * uv
        acc_p2 = acc_p2 + pv * pv
      bv = plsc.load_gather(bpos_v, [uids])
      acc_bi2 = acc_bi2 + bv * bv
      plsc.store_scatter(pscore_v, [uids], T * pd + bv - 1.0)

    def chunk_body(c, accs):
      cps = []
      for j in range(GPC):
        g = c * GPC + j
        cps.append(pltpu.async_copy(qi_hbm.at[nmr_v.at[g]],
                                    nrows_v.at[pl.ds(j * 128, 128)], sem))
        cps.append(pltpu.async_copy(bi_hbm.at[nidx_v.at[pl.ds(g * 128, 128)]],
                                    bneg_v.at[pl.ds(j * 128, 128)], sem))
      for cp in cps:
        cp.wait()

      def grp_body(g, accs2):
        acc_sq, acc_n2, acc_bj2 = accs2
        rows = lanes + g * 16
        p_local = c * ROWS_PC + g * 16 + lanes
        uids = p_local // NNEG
        it = plsc.load_gather(nidx_v, [p_local])
        nbase = lax.mul(lax.bitwise_and(it, 3), 32)
        ubase = plsc.load_gather(ucb_v, [uids])
        dot = f32z
        an2 = f32z
        for d in range(D):
          nv = plsc.load_gather(nrows_v, [rows, nbase + d])
          ut = plsc.load_gather(urows_v, [uids, ubase + d])
          dot = dot + ut * nv
          an2 = an2 + nv * nv
        bj = plsc.load_gather(bneg_v, [rows])
        cs = plsc.load_gather(pscore_v, [uids])
        diff = cs - (T * dot + bj)
        return (acc_sq + diff * diff, acc_n2 + an2, acc_bj2 + bj * bj)

      return lax.fori_loop(0, PG_PC, grp_body, accs)

    acc_sq, acc_n2, acc_bj2 = lax.fori_loop(
        0, NCHUNK, chunk_body, (f32z, f32z, f32z))

    total = (acc_sq * (1.0 / float(B * NNEG))
             + BATA * (acc_u2 + acc_p2 + acc_n2)
             + LAMDA * (acc_bi2 + acc_bj2))
    out_v[0, :] = total
    pltpu.sync_copy(out_v, out_hbm.at[w])

  return _sc_fused


def _tc_reduce_body(x_ref, out_ref):
  out_ref[0, 0] = jnp.sum(x_ref[...])


@functools.cache
def _build_tc_reduce():
  return pl.pallas_call(
      _tc_reduce_body,
      in_specs=[pl.BlockSpec((NW, 16), lambda: (0, 0))],
      out_specs=pl.BlockSpec(memory_space=pltpu.SMEM),
      out_shape=jax.ShapeDtypeStruct((1, 1), jnp.float32),
  )


def kernel(users, pos_items, neg_items, pu, qi, bi):
  neg1 = neg_items.reshape(-1)
  pu4 = pu.reshape(pu.shape[0] // 4, 128)
  qi4 = _build_sc_relayout()(qi.T)
  bif = bi.reshape(-1)
  partials = _build_sc_fused()(users, pos_items, neg1, pu4, qi4, bif)
  res = _build_tc_reduce()(partials.reshape(NW, 16))
  return res[0, 0]


# speedup vs baseline: 1.4754x; 1.4754x over previous
import functools
import jax
import jax.numpy as jnp
from jax import lax
from jax.experimental import pallas as pl
from jax.experimental.pallas import tpu as pltpu
from jax.experimental.pallas import tpu_sc as plsc

B = 4096
NNEG = 50
D = 32
BATA = 0.01
LAMDA = 0.01
T = float(B - 1) ** -0.5

NC = 2
NS = 16
NW = NC * NS
UPW = B // NW
NPW = UPW * NNEG
GPW = NPW // 128
NCHUNK = 5
GPC = GPW // NCHUNK
ROWS_PC = GPC * 128
PG_PC = ROWS_PC // 16


@functools.cache
def _build_sc_fused():
  mesh = plsc.VectorSubcoreMesh(core_axis_name="c", subcore_axis_name="s")

  @functools.partial(
      pl.kernel,
      out_type=jax.ShapeDtypeStruct((NW, 1, 16), jnp.float32),
      mesh=mesh,
      scratch_types=(
          pltpu.VMEM((1, UPW), jnp.int32),
          pltpu.VMEM((1, UPW), jnp.int32),
          pltpu.VMEM((GPW, 128), jnp.int32),
          pltpu.VMEM((UPW, D), jnp.float32),
          pltpu.VMEM((UPW, D), jnp.float32),
          pltpu.VMEM((ROWS_PC, D), jnp.float32),
          pltpu.VMEM((1, UPW), jnp.float32),
          pltpu.VMEM((ROWS_PC,), jnp.float32),
          pltpu.VMEM((1, UPW), jnp.float32),
          pltpu.VMEM((1, 16), jnp.float32),
          pltpu.SemaphoreType.DMA,
      ),
      compiler_params=pltpu.CompilerParams(
          use_tc_tiling_on_sc=False, needs_layout_passes=False),
  )
  def _sc_fused(users_hbm, pos_hbm, neg_hbm, pu_hbm, qi_hbm, bi_hbm,
                out_hbm,
                uidx_v, pidx_v, nidx_v, urows_v, prows_v, nrows_v,
                bpos_v, bneg_v, pscore_v, out_v, sem):
    w = lax.axis_index("s") * NC + lax.axis_index("c")
    pltpu.sync_copy(users_hbm.at[w], uidx_v.at[0])
    pltpu.sync_copy(pos_hbm.at[w], pidx_v.at[0])
    pltpu.sync_copy(neg_hbm.at[w], nidx_v)
    cp_u = pltpu.async_copy(pu_hbm.at[uidx_v.at[0]], urows_v, sem)
    cp_p = pltpu.async_copy(qi_hbm.at[pidx_v.at[0]], prows_v, sem)
    cp_b = pltpu.async_copy(bi_hbm.at[pidx_v.at[0]], bpos_v.at[0], sem)
    cp_u.wait()
    cp_p.wait()
    cp_b.wait()

    lanes = jnp.arange(16, dtype=jnp.int32)
    zero16 = jnp.zeros((16,), jnp.int32)
    f32z = jnp.zeros((16,), jnp.float32)

    acc_u2 = f32z
    acc_p2 = f32z
    acc_bi2 = f32z
    for g in range(UPW // 16):
      uids = lanes + (g * 16)
      pd = f32z
      for d in range(D):
        cold = jnp.full((16,), d, jnp.int32)
        uv = plsc.load_gather(urows_v, [uids, cold])
        pv = plsc.load_gather(prows_v, [uids, cold])
        pd = pd + uv * pv
        acc_u2 = acc_u2 + uv * uv
        acc_p2 = acc_p2 + pv * pv
      bv = plsc.load_gather(bpos_v, [zero16, uids])
      acc_bi2 = acc_bi2 + bv * bv
      plsc.store_scatter(pscore_v, [zero16, uids], T * pd + bv - 1.0)

    def chunk_body(c, accs):
      cps = []
      for j in range(GPC):
        g = c * GPC + j
        cps.append(pltpu.async_copy(qi_hbm.at[nidx_v.at[g]],
                                    nrows_v.at[pl.ds(j * 128, 128)], sem))
        cps.append(pltpu.async_copy(bi_hbm.at[nidx_v.at[g]],
                                    bneg_v.at[pl.ds(j * 128, 128)], sem))
      for cp in cps:
        cp.wait()

      def grp_body(g, accs2):
        acc_sq, acc_n2, acc_bj2 = accs2
        rows = lanes + g * 16
        p_local = c * ROWS_PC + g * 16 + lanes
        uids = p_local // NNEG
        dot = f32z
        an2 = f32z
        for d in range(D):
          cold = jnp.full((16,), d, jnp.int32)
          nv = plsc.load_gather(nrows_v, [rows, cold])
          ut = plsc.load_gather(urows_v, [uids, cold])
          dot = dot + ut * nv
          an2 = an2 + nv * nv
        bj = plsc.load_gather(bneg_v, [rows])
        cs = plsc.load_gather(pscore_v, [zero16, uids])
        diff = cs - (T * dot + bj)
        return (acc_sq + diff * diff, acc_n2 + an2, acc_bj2 + bj * bj)

      return lax.fori_loop(0, PG_PC, grp_body, accs)

    acc_sq, acc_n2, acc_bj2 = lax.fori_loop(
        0, NCHUNK, chunk_body, (f32z, f32z, f32z))

    total = (acc_sq * (1.0 / float(B * NNEG))
             + BATA * (acc_u2 + acc_p2 + acc_n2)
             + LAMDA * (acc_bi2 + acc_bj2))
    out_v[0, :] = total
    pltpu.sync_copy(out_v, out_hbm.at[w])

  return _sc_fused


def _tc_reduce_body(x_ref, out_ref):
  out_ref[0, 0] = jnp.sum(x_ref[...])


@functools.cache
def _build_tc_reduce():
  return pl.pallas_call(
      _tc_reduce_body,
      in_specs=[pl.BlockSpec((NW, 16), lambda: (0, 0))],
      out_specs=pl.BlockSpec(memory_space=pltpu.SMEM),
      out_shape=jax.ShapeDtypeStruct((1, 1), jnp.float32),
  )


def kernel(users, pos_items, neg_items, pu, qi, bi):
  users2 = users.reshape(NW, UPW)
  pos2 = pos_items.reshape(NW, UPW)
  neg3 = neg_items.reshape(NW, GPW, 128)
  bif = bi.reshape(-1)
  partials = _build_sc_fused()(users2, pos2, neg3, pu, qi, bif)
  res = _build_tc_reduce()(partials.reshape(NW, 16))
  return res[0, 0]
